# Initial kernel scaffold; baseline (speedup 1.0000x reference)
import functools

import jax
import jax.numpy as jnp
from jax import lax
from jax.experimental import pallas as pl
from jax.experimental.pallas import tpu as pltpu
from jax.experimental.pallas import tpu_sc as plsc

N = 10000
E = 320000
D = 128
NEG_SLOPE = 0.01

NC = 2
NS = 16
L = 16
NW = NC * NS
EP = E // NW
K = 80
NCHUNK = EP // K
NPAD = 10240
RPT = NPAD // NS
ZB = RPT // K

_sc_mesh = plsc.VectorSubcoreMesh(core_axis_name="c", subcore_axis_name="s")


def _deg_body(src_hbm, dst_hbm, odeg_hbm, ideg_hbm, dstp_hbm,
              src_v, dst_v, dstp_v, odeg_v, ideg_v):
    wid = lax.axis_index("s") * NC + lax.axis_index("c")
    pltpu.sync_copy(src_hbm.at[wid], src_v)
    pltpu.sync_copy(dst_hbm.at[wid], dst_v)

    zero16 = jnp.zeros((L,), jnp.float32)

    def zero_body(i, carry):
        odeg_v[pl.ds(i * L, L)] = zero16
        ideg_v[pl.ds(i * L, L)] = zero16
        return carry

    lax.fori_loop(0, N // L, zero_body, 0)

    ones16 = jnp.ones((L,), jnp.float32)
    trash16 = jnp.full((L,), N, jnp.int32)

    def body(i, carry):
        s16 = src_v[pl.ds(i * L, L)]
        d16 = dst_v[pl.ds(i * L, L)]
        m = s16 != d16
        plsc.addupdate_scatter(odeg_v, [s16], ones16, mask=m)
        plsc.addupdate_scatter(ideg_v, [d16], ones16, mask=m)
        dstp_v[pl.ds(i * L, L)] = jnp.where(m, d16, trash16)
        return carry

    lax.fori_loop(0, EP // L, body, 0)

    pltpu.sync_copy(odeg_v, odeg_hbm.at[wid])
    pltpu.sync_copy(ideg_v, ideg_hbm.at[wid])
    pltpu.sync_copy(dstp_v, dstp_hbm.at[wid])


_deg_call = functools.partial(
    pl.kernel,
    out_type=(
        jax.ShapeDtypeStruct((NW, N), jnp.float32),
        jax.ShapeDtypeStruct((NW, N), jnp.float32),
        jax.ShapeDtypeStruct((NW, EP), jnp.int32),
    ),
    mesh=_sc_mesh,
    scratch_types=[
        pltpu.VMEM((EP,), jnp.int32),
        pltpu.VMEM((EP,), jnp.int32),
        pltpu.VMEM((EP,), jnp.int32),
        pltpu.VMEM((N,), jnp.float32),
        pltpu.VMEM((N,), jnp.float32),
    ],
)(_deg_body)


def _spmm_body(msg_hbm, srcr_hbm, dstr_hbm, out_hbm,
               sidx_v, didx_v, rows_v, accum_sh):
    cid = lax.axis_index("c")
    sid = lax.axis_index("s")
    wid = sid * NC + cid

    pltpu.sync_copy(srcr_hbm.at[wid], sidx_v)
    pltpu.sync_copy(dstr_hbm.at[wid], didx_v)

    zero16 = jnp.zeros((L,), jnp.float32)

    def zrow(i, carry):
        r = i // (D // L)
        c = i % (D // L)
        rows_v[r, pl.ds(c * L, L)] = zero16
        return carry

    lax.fori_loop(0, K * D // L, zrow, 0)

    def zcopy(j, carry):
        pltpu.sync_copy(rows_v, accum_sh.at[pl.ds(sid * RPT + j * K, K)])
        return carry

    lax.fori_loop(0, ZB, zcopy, 0)
    plsc.subcore_barrier()

    def chunk(j, carry):
        pltpu.sync_copy(msg_hbm.at[sidx_v.at[j]], rows_v)
        pltpu.sync_copy(rows_v, accum_sh.at[didx_v.at[j]], add=True)
        return carry

    lax.fori_loop(0, NCHUNK, chunk, 0)
    plsc.subcore_barrier()

    pltpu.sync_copy(accum_sh.at[pl.ds(sid * RPT, RPT)], out_hbm.at[cid, sid])


_spmm_call = functools.partial(
    pl.kernel,
    out_type=jax.ShapeDtypeStruct((NC, NS, RPT, D), jnp.float32),
    mesh=_sc_mesh,
    scratch_types=[
        pltpu.VMEM((NCHUNK, K), jnp.int32),
        pltpu.VMEM((NCHUNK, K), jnp.int32),
        pltpu.VMEM((K, D), jnp.float32),
        pltpu.VMEM_SHARED((NPAD, D), jnp.float32),
    ],
)(_spmm_body)


def _prep_body(odeg_ref, ideg_ref, nsrc_ref, ndst_ref):
    od = jnp.sum(odeg_ref[...], axis=0, keepdims=True) + 1.0
    idg = jnp.sum(ideg_ref[...], axis=0, keepdims=True) + 1.0
    nsrc_ref[...] = lax.rsqrt(od)
    ndst_ref[...] = lax.rsqrt(idg)


def _prep(odeg_p, ideg_p):
    return pl.pallas_call(
        _prep_body,
        out_shape=(
            jax.ShapeDtypeStruct((1, N), jnp.float32),
            jax.ShapeDtypeStruct((1, N), jnp.float32),
        ),
    )(odeg_p, ideg_p)


BLK = 1250
GRID = N // BLK


def _scale_body(x_ref, s_ref, o_ref):
    o_ref[...] = x_ref[...] * s_ref[...]


def _scale(x, s_col):
    return pl.pallas_call(
        _scale_body,
        grid=(GRID,),
        in_specs=[
            pl.BlockSpec((BLK, D), lambda i: (i, 0)),
            pl.BlockSpec((BLK, 1), lambda i: (i, 0)),
        ],
        out_specs=pl.BlockSpec((BLK, D), lambda i: (i, 0)),
        out_shape=jax.ShapeDtypeStruct((N, D), jnp.float32),
    )(x, s_col)


def _layer_body(p0_ref, p1_ref, msg_ref, ndst_ref, s_ref, w_ref, b_ref, o_ref):
    t = (p0_ref[0] + p1_ref[0] + msg_ref[...]) * ndst_ref[...]
    h = jnp.dot(t, w_ref[...], preferred_element_type=jnp.float32) + b_ref[...]
    y = jnp.where(h >= 0.0, h, h * NEG_SLOPE)
    o_ref[...] = y * s_ref[...]


def _layer(partials, msg, ndst_col, s_col, w, b_row):
    return pl.pallas_call(
        _layer_body,
        grid=(GRID,),
        in_specs=[
            pl.BlockSpec((1, BLK, D), lambda i: (0, i, 0)),
            pl.BlockSpec((1, BLK, D), lambda i: (1, i, 0)),
            pl.BlockSpec((BLK, D), lambda i: (i, 0)),
            pl.BlockSpec((BLK, 1), lambda i: (i, 0)),
            pl.BlockSpec((BLK, 1), lambda i: (i, 0)),
            pl.BlockSpec((D, D), lambda i: (0, 0)),
            pl.BlockSpec((1, D), lambda i: (0, 0)),
        ],
        out_specs=pl.BlockSpec((BLK, D), lambda i: (i, 0)),
        out_shape=jax.ShapeDtypeStruct((N, D), jnp.float32),
    )(partials, partials, msg, ndst_col, s_col, w, b_row)


def kernel(in_feat, edge_index, W0, b0, W1, b1):
    src = edge_index[0].reshape(NW, EP)
    dst = edge_index[1].reshape(NW, EP)

    odeg_p, ideg_p, dstp = _deg_call(src, dst)
    nsrc_r, ndst_r = _prep(odeg_p, ideg_p)
    nsrc_c = nsrc_r.reshape(N, 1)
    ndst_c = ndst_r.reshape(N, 1)
    ones_c = jnp.ones((N, 1), jnp.float32)

    msg0 = _scale(in_feat, nsrc_c)

    srcr = src.reshape(NW, NCHUNK, K)
    dstr = dstp.reshape(NW, NCHUNK, K)

    part0 = _spmm_call(msg0, srcr, dstr).reshape(NC, NPAD, D)
    msg1 = _layer(part0, msg0, ndst_c, nsrc_c, W0.astype(jnp.float32),
                  b0.reshape(1, D))
    part1 = _spmm_call(msg1, srcr, dstr).reshape(NC, NPAD, D)
    out = _layer(part1, msg1, ndst_c, ones_c, W1.astype(jnp.float32),
                 b1.reshape(1, D))
    return out


# speedup vs baseline: 7.7489x; 7.7489x over previous
import functools

import jax
import jax.numpy as jnp
from jax import lax
from jax.experimental import pallas as pl
from jax.experimental.pallas import tpu as pltpu
from jax.experimental.pallas import tpu_sc as plsc

N = 10000
E = 320000
D = 128
NEG_SLOPE = 0.01

NC = 2
NS = 16
L = 16
NW = NC * NS
EP = E // NW
K = 80
NCHUNK = EP // K
NPAD = 10240
RPT = NPAD // NS
ZB = RPT // K

_sc_mesh = plsc.VectorSubcoreMesh(core_axis_name="c", subcore_axis_name="s")


def _deg_body(src_hbm, dst_hbm, odeg_hbm, ideg_hbm, dstp_hbm,
              src_v, dst_v, dstp_v, odeg_v, ideg_v):
    wid = lax.axis_index("s") * NC + lax.axis_index("c")
    pltpu.sync_copy(src_hbm.at[wid], src_v)
    pltpu.sync_copy(dst_hbm.at[wid], dst_v)

    zero16 = jnp.zeros((L,), jnp.float32)

    def zero_body(i, carry):
        odeg_v[pl.ds(i * L, L)] = zero16
        ideg_v[pl.ds(i * L, L)] = zero16
        return carry

    lax.fori_loop(0, N // L, zero_body, 0)

    ones16 = jnp.ones((L,), jnp.float32)
    trash16 = jnp.full((L,), N, jnp.int32)

    def body(i, carry):
        s16 = src_v[pl.ds(i * L, L)]
        d16 = dst_v[pl.ds(i * L, L)]
        m = s16 != d16
        plsc.addupdate_scatter(odeg_v, [s16], ones16, mask=m)
        plsc.addupdate_scatter(ideg_v, [d16], ones16, mask=m)
        dstp_v[pl.ds(i * L, L)] = jnp.where(m, d16, trash16)
        return carry

    lax.fori_loop(0, EP // L, body, 0)

    pltpu.sync_copy(odeg_v, odeg_hbm.at[wid])
    pltpu.sync_copy(ideg_v, ideg_hbm.at[wid])
    pltpu.sync_copy(dstp_v, dstp_hbm.at[wid])


_deg_call = functools.partial(
    pl.kernel,
    out_type=(
        jax.ShapeDtypeStruct((NW, N), jnp.float32),
        jax.ShapeDtypeStruct((NW, N), jnp.float32),
        jax.ShapeDtypeStruct((NW, EP), jnp.int32),
    ),
    mesh=_sc_mesh,
    scratch_types=[
        pltpu.VMEM((EP,), jnp.int32),
        pltpu.VMEM((EP,), jnp.int32),
        pltpu.VMEM((EP,), jnp.int32),
        pltpu.VMEM((N,), jnp.float32),
        pltpu.VMEM((N,), jnp.float32),
    ],
    compiler_params=pltpu.CompilerParams(needs_layout_passes=False),
)(_deg_body)


def _spmm_body(msg_hbm, srcr_hbm, dstr_hbm, out_hbm,
               sidx_v, didx_v, rows_v, accum_sh):
    cid = lax.axis_index("c")
    sid = lax.axis_index("s")
    wid = sid * NC + cid

    pltpu.sync_copy(srcr_hbm.at[wid], sidx_v)
    pltpu.sync_copy(dstr_hbm.at[wid], didx_v)

    zero16 = jnp.zeros((L,), jnp.float32)

    def zrow(i, carry):
        r = i // (D // L)
        c = i % (D // L)
        rows_v[r, pl.ds(c * L, L)] = zero16
        return carry

    lax.fori_loop(0, K * D // L, zrow, 0)

    def zcopy(j, carry):
        pltpu.sync_copy(rows_v, accum_sh.at[pl.ds(sid * RPT + j * K, K)])
        return carry

    lax.fori_loop(0, ZB, zcopy, 0)
    plsc.subcore_barrier()

    def chunk(j, carry):
        pltpu.sync_copy(msg_hbm.at[sidx_v.at[j]], rows_v)
        pltpu.sync_copy(rows_v, accum_sh.at[didx_v.at[j]], add=True)
        return carry

    lax.fori_loop(0, NCHUNK, chunk, 0)
    plsc.subcore_barrier()

    pltpu.sync_copy(accum_sh.at[pl.ds(sid * RPT, RPT)], out_hbm.at[cid, sid])


_spmm_call = functools.partial(
    pl.kernel,
    out_type=jax.ShapeDtypeStruct((NC, NS, RPT, D), jnp.float32),
    mesh=_sc_mesh,
    scratch_types=[
        pltpu.VMEM((NCHUNK, K), jnp.int32),
        pltpu.VMEM((NCHUNK, K), jnp.int32),
        pltpu.VMEM((K, D), jnp.float32),
        pltpu.VMEM_SHARED((NPAD, D), jnp.float32),
    ],
)(_spmm_body)


def _prep_body(odeg_ref, ideg_ref, nsrc_ref, ndst_ref):
    od = jnp.sum(odeg_ref[...], axis=0, keepdims=True) + 1.0
    idg = jnp.sum(ideg_ref[...], axis=0, keepdims=True) + 1.0
    nsrc_ref[...] = lax.rsqrt(od)
    ndst_ref[...] = lax.rsqrt(idg)


def _prep(odeg_p, ideg_p):
    return pl.pallas_call(
        _prep_body,
        out_shape=(
            jax.ShapeDtypeStruct((1, N), jnp.float32),
            jax.ShapeDtypeStruct((1, N), jnp.float32),
        ),
    )(odeg_p, ideg_p)


BLK = 1000
GRID = N // BLK


def _scale_body(x_ref, s_ref, o_ref):
    o_ref[...] = x_ref[...] * s_ref[...]


def _scale(x, s_col):
    return pl.pallas_call(
        _scale_body,
        grid=(GRID,),
        in_specs=[
            pl.BlockSpec((BLK, D), lambda i: (i, 0)),
            pl.BlockSpec((BLK, 1), lambda i: (i, 0)),
        ],
        out_specs=pl.BlockSpec((BLK, D), lambda i: (i, 0)),
        out_shape=jax.ShapeDtypeStruct((N, D), jnp.float32),
    )(x, s_col)


def _layer_body(p0_ref, p1_ref, msg_ref, ndst_ref, s_ref, w_ref, b_ref, o_ref):
    t = (p0_ref[0] + p1_ref[0] + msg_ref[...]) * ndst_ref[...]
    h = jnp.dot(t, w_ref[...], preferred_element_type=jnp.float32) + b_ref[...]
    y = jnp.where(h >= 0.0, h, h * NEG_SLOPE)
    o_ref[...] = y * s_ref[...]


def _layer(partials, msg, ndst_col, s_col, w, b_row):
    return pl.pallas_call(
        _layer_body,
        grid=(GRID,),
        in_specs=[
            pl.BlockSpec((1, BLK, D), lambda i: (0, i, 0)),
            pl.BlockSpec((1, BLK, D), lambda i: (1, i, 0)),
            pl.BlockSpec((BLK, D), lambda i: (i, 0)),
            pl.BlockSpec((BLK, 1), lambda i: (i, 0)),
            pl.BlockSpec((BLK, 1), lambda i: (i, 0)),
            pl.BlockSpec((D, D), lambda i: (0, 0)),
            pl.BlockSpec((1, D), lambda i: (0, 0)),
        ],
        out_specs=pl.BlockSpec((BLK, D), lambda i: (i, 0)),
        out_shape=jax.ShapeDtypeStruct((N, D), jnp.float32),
    )(partials, partials, msg, ndst_col, s_col, w, b_row)


def kernel(in_feat, edge_index, W0, b0, W1, b1):
    src = edge_index[0].reshape(NW, EP)
    dst = edge_index[1].reshape(NW, EP)

    odeg_p, ideg_p, dstp = _deg_call(src, dst)
    nsrc_r, ndst_r = _prep(odeg_p, ideg_p)
    nsrc_c = nsrc_r.reshape(N, 1)
    ndst_c = ndst_r.reshape(N, 1)
    ones_c = jnp.ones((N, 1), jnp.float32)

    msg0 = _scale(in_feat, nsrc_c)

    srcr = src.reshape(NW, NCHUNK, K)
    dstr = dstp.reshape(NW, NCHUNK, K)

    part0 = _spmm_call(msg0, srcr, dstr).reshape(NC, NPAD, D)
    msg1 = _layer(part0, msg0, ndst_c, nsrc_c, W0.astype(jnp.float32),
                  b0.reshape(1, D))
    part1 = _spmm_call(msg1, srcr, dstr).reshape(NC, NPAD, D)
    out = _layer(part1, msg1, ndst_c, ones_c, W1.astype(jnp.float32),
                 b1.reshape(1, D))
    return out
